# baseline (speedup 1.0000x reference)
import jax
import jax.numpy as jnp
from jax.experimental import pallas as pl
from jax.experimental.pallas import tpu as pltpu

_BM_TARGET = 400


def _pick_block(n, target):
    if n % target == 0:
        return target
    best = 1
    for d in range(1, min(n, target) + 1):
        if n % d == 0:
            best = d
    return best


def _layer1_body(adj_ref, fk_ref, fs_ref, w1_ref, b1_ref,
                 h_ref, hb_ref, deg_ref, mask_ref, *, f):
    a = adj_ref[...]
    ab = a.astype(jnp.bfloat16)
    ai = a.astype(jnp.int32)
    nw = ai.shape[1] // 8
    w = ai[:, :nw]
    for b in range(1, 8):
        w = w | (ai[:, b * nw:(b + 1) * nw] << b)
    mask_ref[...] = w.astype(jnp.int8)
    deg = jnp.maximum(jnp.sum(a, axis=1, keepdims=True), 1.0)
    acc = jnp.dot(ab, fk_ref[...], preferred_element_type=jnp.float32)
    agg = acc / deg
    out = (jnp.dot(fs_ref[...], w1_ref[:f, :],
                   preferred_element_type=jnp.float32)
           + jnp.dot(agg, w1_ref[f:, :], preferred_element_type=jnp.float32)
           + b1_ref[...])
    nrm = jnp.sqrt(jnp.maximum(
        jnp.sum(out * out, axis=1, keepdims=True), 1e-12))
    h = jnp.maximum(out / nrm, 0.0)
    h_ref[...] = h
    hb_ref[...] = h.astype(jnp.bfloat16)
    deg_ref[...] = deg


def _layer2_body(m_ref, hk_ref, hs_ref, deg_ref, w2_ref, b2_ref,
                 o_ref, *, nh):
    w = m_ref[...].astype(jnp.int32)
    m = jnp.concatenate(
        [((w >> b) & 1).astype(jnp.bfloat16) for b in range(8)], axis=1)
    acc = jnp.dot(m, hk_ref[...], preferred_element_type=jnp.float32)
    agg = acc / deg_ref[...]
    out = (jnp.dot(hs_ref[...], w2_ref[:nh, :],
                   preferred_element_type=jnp.float32)
           + jnp.dot(agg, w2_ref[nh:, :], preferred_element_type=jnp.float32)
           + b2_ref[...])
    nrm = jnp.sqrt(jnp.maximum(
        jnp.sum(out * out, axis=1, keepdims=True), 1e-12))
    o = out / nrm
    mx = jnp.max(o, axis=1, keepdims=True)
    e = jnp.exp(o - mx)
    o_ref[...] = e / jnp.sum(e, axis=1, keepdims=True)


def kernel(feats, adj, W1, b1, W2, b2):
    import functools
    n, f = feats.shape
    nhid = W1.shape[1]
    ncls = W2.shape[1]
    bm = _pick_block(n, _BM_TARGET)
    gm = n // bm
    b1r = b1.reshape(1, nhid)
    b2r = b2.reshape(1, ncls)

    h, hb, deg, mask = pl.pallas_call(
        functools.partial(_layer1_body, f=f),
        grid=(gm,),
        in_specs=[
            pl.BlockSpec((bm, n), lambda i: (i, 0)),
            pl.BlockSpec((n, f), lambda i: (0, 0)),
            pl.BlockSpec((bm, f), lambda i: (i, 0)),
            pl.BlockSpec((2 * f, nhid), lambda i: (0, 0)),
            pl.BlockSpec((1, nhid), lambda i: (0, 0)),
        ],
        out_specs=[
            pl.BlockSpec((bm, nhid), lambda i: (i, 0)),
            pl.BlockSpec((bm, nhid), lambda i: (i, 0)),
            pl.BlockSpec((bm, 1), lambda i: (i, 0)),
            pl.BlockSpec((bm, n // 8), lambda i: (i, 0)),
        ],
        out_shape=[
            jax.ShapeDtypeStruct((n, nhid), jnp.float32),
            jax.ShapeDtypeStruct((n, nhid), jnp.bfloat16),
            jax.ShapeDtypeStruct((n, 1), jnp.float32),
            jax.ShapeDtypeStruct((n, n // 8), jnp.int8),
        ],
        compiler_params=pltpu.CompilerParams(
            dimension_semantics=("arbitrary",)),
    )(adj, feats.astype(jnp.bfloat16), feats, W1, b1r)

    o = pl.pallas_call(
        functools.partial(_layer2_body, nh=nhid),
        grid=(gm,),
        in_specs=[
            pl.BlockSpec((bm, n // 8), lambda i: (i, 0)),
            pl.BlockSpec((n, nhid), lambda i: (0, 0)),
            pl.BlockSpec((bm, nhid), lambda i: (i, 0)),
            pl.BlockSpec((bm, 1), lambda i: (i, 0)),
            pl.BlockSpec((2 * nhid, ncls), lambda i: (0, 0)),
            pl.BlockSpec((1, ncls), lambda i: (0, 0)),
        ],
        out_specs=pl.BlockSpec((bm, ncls), lambda i: (i, 0)),
        out_shape=jax.ShapeDtypeStruct((n, ncls), jnp.float32),
        compiler_params=pltpu.CompilerParams(
            dimension_semantics=("arbitrary",)),
    )(mask, hb, h, deg, W2, b2r)
    return o


# speedup vs baseline: 1.0381x; 1.0381x over previous
import functools

import jax
import jax.numpy as jnp
from jax.experimental import pallas as pl
from jax.experimental.pallas import tpu as pltpu

_BM_TARGET = 400


def _pick_block(n, target):
    if n % target == 0:
        return target
    best = 1
    for d in range(1, min(n, target) + 1):
        if n % d == 0:
            best = d
    return best


def _layer1_body(adj_ref, fk_ref, fs_ref, w1_ref, b1_ref,
                 hb_ref, deg_ref, mask_ref, *, f, nw):
    a = adj_ref[...]
    n = a.shape[1]
    ab = a.astype(jnp.bfloat16)
    wbf = None
    for b in range(8):
        lo = b * nw
        if lo >= n:
            continue
        hi = min(lo + nw, n)
        s = ab[:, lo:hi]
        if hi - lo < nw:
            s = jnp.pad(s, ((0, 0), (0, nw - (hi - lo))))
        s = s * jnp.bfloat16(2 ** b) if b else s
        wbf = s if wbf is None else wbf + s
    mask_ref[...] = wbf.astype(jnp.int32).astype(jnp.int8)
    deg = jnp.maximum(jnp.sum(a, axis=1, keepdims=True), 1.0)
    acc = jnp.dot(ab, fk_ref[...], preferred_element_type=jnp.float32)
    agg = (acc / deg).astype(jnp.bfloat16)
    out = (jnp.dot(fs_ref[...], w1_ref[:f, :],
                   preferred_element_type=jnp.float32)
           + jnp.dot(agg, w1_ref[f:, :], preferred_element_type=jnp.float32)
           + b1_ref[...])
    nrm = jnp.sqrt(jnp.maximum(
        jnp.sum(out * out, axis=1, keepdims=True), 1e-12))
    hb_ref[...] = (jnp.maximum(out, 0.0) / nrm).astype(jnp.bfloat16)
    deg_ref[...] = deg


def _layer2_body(m_ref, hk_ref, hs_ref, deg_ref, w2_ref, b2_ref,
                 o_ref, *, nh):
    wi = m_ref[...].astype(jnp.int32)
    m = jnp.concatenate(
        [((wi >> b) & 1).astype(jnp.bfloat16) for b in range(8)], axis=1)
    acc = jnp.dot(m, hk_ref[...], preferred_element_type=jnp.float32)
    agg = (acc / deg_ref[...]).astype(jnp.bfloat16)
    out = (jnp.dot(hs_ref[...], w2_ref[:nh, :],
                   preferred_element_type=jnp.float32)
           + jnp.dot(agg, w2_ref[nh:, :], preferred_element_type=jnp.float32)
           + b2_ref[...])
    nrm = jnp.sqrt(jnp.maximum(
        jnp.sum(out * out, axis=1, keepdims=True), 1e-12))
    o = out / nrm
    mx = jnp.max(o, axis=1, keepdims=True)
    e = jnp.exp(o - mx)
    o_ref[...] = e / jnp.sum(e, axis=1, keepdims=True)


def kernel(feats, adj, W1, b1, W2, b2):
    n, f = feats.shape
    nhid = W1.shape[1]
    ncls = W2.shape[1]
    bm = _pick_block(n, _BM_TARGET)
    gm = n // bm
    nw = (((n + 7) // 8 + 127) // 128) * 128
    b1r = b1.reshape(1, nhid)
    b2r = b2.reshape(1, ncls)
    fb = feats.astype(jnp.bfloat16)
    w1b = W1.astype(jnp.bfloat16)
    w2b = W2.astype(jnp.bfloat16)

    hb, deg, mask = pl.pallas_call(
        functools.partial(_layer1_body, f=f, nw=nw),
        grid=(gm,),
        in_specs=[
            pl.BlockSpec((bm, n), lambda i: (i, 0)),
            pl.BlockSpec((n, f), lambda i: (0, 0)),
            pl.BlockSpec((bm, f), lambda i: (i, 0)),
            pl.BlockSpec((2 * f, nhid), lambda i: (0, 0)),
            pl.BlockSpec((1, nhid), lambda i: (0, 0)),
        ],
        out_specs=[
            pl.BlockSpec((bm, nhid), lambda i: (i, 0)),
            pl.BlockSpec((bm, 1), lambda i: (i, 0)),
            pl.BlockSpec((bm, nw), lambda i: (i, 0)),
        ],
        out_shape=[
            jax.ShapeDtypeStruct((n, nhid), jnp.bfloat16),
            jax.ShapeDtypeStruct((n, 1), jnp.float32),
            jax.ShapeDtypeStruct((n, nw), jnp.int8),
        ],
        compiler_params=pltpu.CompilerParams(
            dimension_semantics=("arbitrary",)),
    )(adj, fb, fb, w1b, b1r)

    hkp = jnp.pad(hb, ((0, 8 * nw - n), (0, 0)))

    o = pl.pallas_call(
        functools.partial(_layer2_body, nh=nhid),
        grid=(gm,),
        in_specs=[
            pl.BlockSpec((bm, nw), lambda i: (i, 0)),
            pl.BlockSpec((8 * nw, nhid), lambda i: (0, 0)),
            pl.BlockSpec((bm, nhid), lambda i: (i, 0)),
            pl.BlockSpec((bm, 1), lambda i: (i, 0)),
            pl.BlockSpec((2 * nhid, ncls), lambda i: (0, 0)),
            pl.BlockSpec((1, ncls), lambda i: (0, 0)),
        ],
        out_specs=pl.BlockSpec((bm, ncls), lambda i: (i, 0)),
        out_shape=jax.ShapeDtypeStruct((n, ncls), jnp.float32),
        compiler_params=pltpu.CompilerParams(
            dimension_semantics=("arbitrary",)),
    )(mask, hkp, hb, deg, w2b, b2r)
    return o
